# baseline (speedup 1.0000x reference)
import functools

import jax
import jax.numpy as jnp
from jax import lax
from jax.experimental import pallas as pl
from jax.experimental.pallas import tpu as pltpu
from jax.experimental.pallas import tpu_sc as plsc

NY, NX = 496, 432
NCELL = NY * NX
P = 100000
C = 64
NT = 32
SBLK = 128
NBLK = NCELL // SBLK
NB_BIG = 53
NB_SMALL = 52
N_BIG = NBLK - NB_SMALL * NT
MAXCELLS = NB_BIG * SBLK
SENT = 1 << 30
CHUNK = 4096
NFULL = P // CHUNK
TAIL = P - NFULL * CHUNK
NBUF = 4
NOUT = 2


def _scatter_body(feats_hbm, coords_hbm, out_hbm,
                  inv_v, coords_v, idx_v, fb_v, rows_v, out_v,
                  csem, gsem, wsem):
    wid = lax.axis_index("s") * 2 + lax.axis_index("c")
    iota16 = lax.iota(jnp.int32, 16)

    base_cell = jnp.where(
        wid < N_BIG,
        wid * (NB_BIG * SBLK),
        N_BIG * (NB_BIG * SBLK) + (wid - N_BIG) * (NB_SMALL * SBLK),
    )
    nb = jnp.where(wid < N_BIG, NB_BIG, NB_SMALL)
    ncells_t = nb * SBLK

    def fire_chunk(c, kb):
        pltpu.async_copy(coords_hbm.at[pl.ds(c * CHUNK, CHUNK)],
                         coords_v.at[kb], csem)

    def wait_chunk(c, kb):
        pltpu.make_async_copy(coords_hbm.at[pl.ds(c * CHUNK, CHUNK)],
                              coords_v.at[kb], csem).wait()

    fire_chunk(0, 0)

    sent_v = jnp.full((16,), SENT, dtype=jnp.int32)

    @plsc.parallel_loop(0, MAXCELLS // 16, 1, unroll=8)
    def _(j):
        inv_v[pl.ds(j * 16, 16)] = sent_v

    zf = jnp.zeros((16,), dtype=jnp.float32)
    for k in range(NBUF):
        for j in range(C // 16):
            rows_v[k, SBLK, pl.ds(j * 16, 16)] = zf

    col2 = jnp.full((16,), 2, dtype=jnp.int32)
    col3 = jnp.full((16,), 3, dtype=jnp.int32)

    def scan_rows(kb, row_base, nrows):
        cref = coords_v.at[kb]

        @plsc.parallel_loop(0, nrows // 16, 1, unroll=8)
        def _(g):
            ridx = g * 16 + iota16
            y = plsc.load_gather(cref, [ridx, col2])
            x = plsc.load_gather(cref, [ridx, col3])
            rel = y * NX + x - base_cell
            m = (rel >= 0) & (rel < ncells_t)
            relc = jnp.where(m, rel, 0)
            plsc.store_scatter(inv_v, [relc], row_base + ridx, mask=m)

    fire_chunk(1, 1)

    def chunk_pair(s, _):
        for cc in range(2):
            c = 2 * s + cc
            wait_chunk(c, cc)
            fire_chunk(c + 2, cc)
            scan_rows(cc, c * CHUNK, CHUNK)
        return 0

    lax.fori_loop(0, NFULL // 2 - 1, chunk_pair, 0)
    wait_chunk(NFULL - 2, 0)
    scan_rows(0, (NFULL - 2) * CHUNK, CHUNK)
    pltpu.async_copy(coords_hbm.at[pl.ds(NFULL * CHUNK, TAIL)],
                     coords_v.at[0, pl.ds(0, TAIL)], csem)
    wait_chunk(NFULL - 1, 1)
    scan_rows(1, (NFULL - 1) * CHUNK, CHUNK)
    pltpu.make_async_copy(coords_hbm.at[pl.ds(NFULL * CHUNK, TAIL)],
                          coords_v.at[0, pl.ds(0, TAIL)], csem).wait()
    scan_rows(0, NFULL * CHUNK, TAIL)

    def prep(g, k):
        cb = g * SBLK
        for j in range(SBLK // 16):
            inv16 = inv_v[pl.ds(cb + j * 16, 16)]
            idx_v[k, pl.ds(j * 16, 16)] = inv16
            rloc = jnp.where(inv16 != SENT, j * 16 + iota16, SBLK)
            fb_v[k, pl.ds(j * 16, 16)] = rloc

    def gather_refs(k):
        return (feats_hbm.at[plsc.Indices(idx_v.at[k], ignored_value=SENT)],
                rows_v.at[k, pl.ds(0, SBLK), :])

    def fire_gather(k):
        src, dst = gather_refs(k)
        pltpu.async_copy(src, dst, gsem)

    def wait_gather(k):
        src, dst = gather_refs(k)
        pltpu.make_async_copy(src, dst, gsem).wait()

    def transpose(k, m):
        rref = rows_v.at[k]
        rlocs = tuple(fb_v[k, pl.ds(j * 16, 16)] for j in range(SBLK // 16))

        @plsc.parallel_loop(0, C, 1, unroll=4)
        def _(c):
            csplat = jnp.full((16,), c, dtype=jnp.int32)
            for j in range(SBLK // 16):
                v = plsc.load_gather(rref, [rlocs[j], csplat])
                out_v[m, c, pl.ds(j * 16, 16)] = v

    def write_refs(g, m):
        col = base_cell + g * SBLK
        return out_v.at[m], out_hbm.at[:, pl.ds(col, SBLK)]

    def fire_write(g, m):
        src, dst = write_refs(g, m)
        pltpu.async_copy(src, dst, wsem)

    def wait_write(g, m):
        src, dst = write_refs(g, m)
        pltpu.make_async_copy(src, dst, wsem).wait()

    for g in range(NBUF):
        prep(g, g)
        fire_gather(g)
    for m in range(NOUT):
        fire_write(m, m)

    def block_quad(s, _):
        for kk in range(NBUF):
            g = NBUF * s + kk
            m = kk % NOUT
            wait_gather(kk)
            wait_write(g - NOUT + NOUT * (g < NOUT), m)
            transpose(kk, m)
            fire_write(g, m)
            prep(g + NBUF, kk)
            fire_gather(kk)
        return 0

    lax.fori_loop(0, NB_SMALL // NBUF - 1, block_quad, 0)
    for kk in range(NBUF):
        g = NB_SMALL - NBUF + kk
        m = kk % NOUT
        wait_gather(kk)
        wait_write(g - NOUT, m)
        transpose(kk, m)
        fire_write(g, m)
    wait_write(NB_SMALL - 2, 0)
    wait_write(NB_SMALL - 1, 1)

    @pl.when(wid < N_BIG)
    def _():
        prep(NB_SMALL, 0)
        fire_gather(0)
        wait_gather(0)
        transpose(0, 0)
        fire_write(NB_SMALL, 0)
        wait_write(NB_SMALL, 0)


@jax.jit
def _pillar_scatter(pillar_feats, coords):
    mesh = plsc.VectorSubcoreMesh(core_axis_name="c", subcore_axis_name="s")
    f = functools.partial(
        pl.kernel,
        out_type=jax.ShapeDtypeStruct((C, NCELL), jnp.float32),
        mesh=mesh,
        scratch_types=[
            pltpu.VMEM((MAXCELLS,), jnp.int32),
            pltpu.VMEM((2, CHUNK, 4), jnp.int32),
            pltpu.VMEM((NBUF, SBLK), jnp.int32),
            pltpu.VMEM((NBUF, SBLK), jnp.int32),
            pltpu.VMEM((NBUF, SBLK + 1, C), jnp.float32),
            pltpu.VMEM((NOUT, C, SBLK), jnp.float32),
            pltpu.SemaphoreType.DMA,
            pltpu.SemaphoreType.DMA,
            pltpu.SemaphoreType.DMA,
        ],
        compiler_params=pltpu.CompilerParams(
            needs_layout_passes=False, use_tc_tiling_on_sc=False
        ),
    )(_scatter_body)
    return f(pillar_feats, coords)


def kernel(pillar_feats, coords):
    canvas_t = _pillar_scatter(pillar_feats, coords)
    return canvas_t.reshape(1, C, NY, NX)
